# Initial kernel scaffold; baseline (speedup 1.0000x reference)
import jax
import jax.numpy as jnp
from jax.experimental import pallas as pl


def kernel(labels, edges_nn, fx, fy, skip0, skip1, skip2, skip3, W_pl0, W_pr0, b_p0, W_nl0, W_nr0, b_n0, W_pl1, W_pr1, b_p1, W_nl1, W_nr1, b_n1, W_pl2, W_pr2, b_p2, W_nl2, W_nr2, b_n2, W_pl3, W_pr3, b_p3, W_nl3, W_nr3, b_n3, bn0_g, bn0_b, bn1_g, bn1_b, bn2_g, bn2_b, bn3_g, bn3_b, W_pw, b_pw):
    raise NotImplementedError("write your pallas kernel here")


# speedup vs baseline: 1.8038x; 1.8038x over previous
import functools
import numpy as np
import jax
import jax.numpy as jnp
from jax import lax
from jax.experimental import pallas as pl
from jax.experimental.pallas import tpu as pltpu
from jax.experimental.pallas import tpu_sc as plsc

B = 4
SSP = 2500
SPAD = 2560
NN = 10000
NNP = 10240
NH = 5120
NAB = 5248
TRASH = 5184
EE = 160000
EEP = 163840
EPT = EEP // 16
ECH = 512
NECH = EPT // ECH
PPT = 16384 // 16
PCH = 256
NPCH = PPT // PCH
RB = 1000
NRB = NN // RB


@functools.cache
def _mesh():
    return plsc.VectorSubcoreMesh(core_axis_name="c", subcore_axis_name="s")


def _up_mat(w):
    xs = np.linspace(0.0, w - 1.0, 128)
    x0 = np.floor(xs).astype(np.int32)
    x1 = np.minimum(x0 + 1, w - 1)
    wx = (xs - x0).astype(np.float32)
    U = np.zeros((128, w), np.float32)
    np.add.at(U, (np.arange(128), x0), 1.0 - wx)
    np.add.at(U, (np.arange(128), x1), wx)
    return jnp.asarray(U)


def _upsample(f, U):
    Bb, C, w, h = f.shape

    def body(f_ref, u_ref, o_ref):
        f3 = f_ref[0]
        Um = u_ref[...]
        t1 = lax.dot_general(Um, f3, (((1,), (1,)), ((), ())),
                             preferred_element_type=jnp.float32)
        t2 = lax.dot_general(t1, Um, (((2,), (1,)), ((), ())),
                             preferred_element_type=jnp.float32)
        o_ref[0] = jnp.transpose(t2, (1, 0, 2)).reshape(C, 128 * 128)

    return pl.pallas_call(
        body,
        grid=(Bb,),
        in_specs=[pl.BlockSpec((1, C, w, h), lambda b: (b, 0, 0, 0)),
                  pl.BlockSpec((128, w), lambda b: (0, 0))],
        out_specs=pl.BlockSpec((1, C, 16384), lambda b: (b, 0, 0)),
        out_shape=jax.ShapeDtypeStruct((Bb, C, 16384), jnp.float32),
    )(f, U)


def _to_pixel_major(cm, Cp):
    Bb, C, _ = cm.shape
    PB = 2048

    def body(x_ref, o_ref):
        x = x_ref[0]
        y = jnp.transpose(x, (1, 0))
        if Cp > C:
            y = jnp.concatenate(
                [y, jnp.zeros((PB, Cp - C), jnp.float32)], axis=1)
        o_ref[...] = y

    return pl.pallas_call(
        body,
        grid=(Bb, 16384 // PB),
        in_specs=[pl.BlockSpec((1, C, PB), lambda b, j: (b, 0, j))],
        out_specs=pl.BlockSpec((PB, Cp), lambda b, j: (b * (16384 // PB) + j, 0)),
        out_shape=jax.ShapeDtypeStruct((Bb * 16384, Cp), jnp.float32),
    )(cm)


def _feat0_pixel_major(s0cm, fx, fy):
    PB = 2048

    def body(s_ref, fx_ref, fy_ref, o_ref):
        j = pl.program_id(1)
        s = jnp.transpose(s_ref[0], (1, 0))
        pix = j * PB + lax.broadcasted_iota(jnp.int32, (PB, 1), 0)
        cx = (pix // 128).astype(jnp.float32) / 127.0
        cy = (pix % 128).astype(jnp.float32) / 127.0
        fxc = jnp.transpose(fx_ref[0], (1, 0))
        fyc = jnp.transpose(fy_ref[0], (1, 0))
        y = jnp.concatenate(
            [cx, cy, fxc, fyc, s, jnp.zeros((PB, 92), jnp.float32)], axis=1)
        o_ref[...] = y

    return pl.pallas_call(
        body,
        grid=(B, 16384 // PB),
        in_specs=[pl.BlockSpec((1, 32, PB), lambda b, j: (b, 0, j)),
                  pl.BlockSpec((1, 1, PB), lambda b, j: (b, 0, j)),
                  pl.BlockSpec((1, 1, PB), lambda b, j: (b, 0, j))],
        out_specs=pl.BlockSpec((PB, 128), lambda b, j: (b * (16384 // PB) + j, 0)),
        out_shape=jax.ShapeDtypeStruct((B * 16384, 128), jnp.float32),
    )(s0cm, fx, fy)


def _edge_prep(edges_nn):
    def body(e_ref, o_ref):
        pad = jnp.full((EEP - EE,), TRASH, jnp.int32)
        zpad = jnp.zeros((EEP - EE,), jnp.int32)
        dst = e_ref[1]
        sgn = e_ref[2]
        neg = sgn == -1
        o_ref[0, :] = jnp.concatenate([e_ref[0], zpad])
        for q, (want_neg, h) in enumerate([(False, 0), (False, 1),
                                           (True, 0), (True, 1)]):
            sel = ((neg == want_neg) & (dst >= h * NH) & (dst < (h + 1) * NH))
            loc = jnp.where(sel, dst - h * NH, TRASH)
            o_ref[1 + q, :] = jnp.concatenate([loc, pad])
        o_ref[5, :] = jnp.zeros((EEP,), jnp.int32)
        o_ref[6, :] = jnp.zeros((EEP,), jnp.int32)
        o_ref[7, :] = jnp.zeros((EEP,), jnp.int32)

    return pl.pallas_call(
        body,
        in_specs=[pl.BlockSpec((3, EE), lambda: (0, 0))],
        out_specs=pl.BlockSpec((8, EEP), lambda: (0, 0)),
        out_shape=jax.ShapeDtypeStruct((8, EEP), jnp.int32),
    )(edges_nn)


def _label_prep(labels):
    def body(l_ref, o_ref):
        b = pl.program_id(0)
        o_ref[...] = l_ref[...] + (b % 2) * SPAD

    return pl.pallas_call(
        body,
        grid=(B,),
        in_specs=[pl.BlockSpec((1, 1, 16384), lambda b: (b, 0, 0))],
        out_specs=pl.BlockSpec((1, 1, 16384), lambda b: (b, 0, 0)),
        out_shape=jax.ShapeDtypeStruct((B, 1, 16384), jnp.int32),
    )(labels.reshape(B, 1, 16384))


def _sc_pool_all(f0, f1, f2, f3a, f3b, labels2):
    outs = [jax.ShapeDtypeStruct((B * SPAD, 128), jnp.float32)] * 6
    scr = [pltpu.VMEM((PCH, 128), jnp.float32),
           pltpu.VMEM((PCH,), jnp.int32),
           pltpu.VMEM_SHARED((2 * SPAD, 128), jnp.float32)]
    zc128 = jnp.zeros((160, 128), jnp.float32)
    ones_px = jnp.ones((PCH, 128), jnp.float32)

    @functools.partial(pl.kernel, mesh=_mesh(), out_type=outs,
                       scratch_types=scr)
    def k(f0_h, f1_h, f2_h, f3a_h, f3b_h, lab_h, zc_h, ones_h,
          s0_h, cnt_h, s1_h, s2_h, s3a_h, s3b_h, rows_v, lab_v, acc):
        c = lax.axis_index("c")
        s = lax.axis_index("s")

        def do_set(feats_h, out_h):
            for b in range(2):
                pltpu.sync_copy(zc_h, acc.at[pl.ds(b * SPAD + s * 160, 160)])
            plsc.subcore_barrier()
            for b in range(2):
                batch = c * 2 + b
                for kk in range(NPCH):
                    pltpu.sync_copy(lab_h.at[(batch * 16 + s) * NPCH + kk],
                                    lab_v)
                    if feats_h is not None:
                        row0 = batch * 16384 + s * PPT + kk * PCH
                        pltpu.sync_copy(feats_h.at[pl.ds(row0, PCH)], rows_v)
                    pltpu.sync_copy(rows_v, acc.at[lab_v], add=True)
            plsc.subcore_barrier()
            for b in range(2):
                batch = c * 2 + b
                pltpu.sync_copy(acc.at[pl.ds(b * SPAD + s * 160, 160)],
                                out_h.at[pl.ds(batch * SPAD + s * 160, 160)])

        do_set(f0_h, s0_h)
        pltpu.sync_copy(ones_h, rows_v)
        do_set(None, cnt_h)
        do_set(f1_h, s1_h)
        do_set(f2_h, s2_h)
        do_set(f3a_h, s3a_h)
        do_set(f3b_h, s3b_h)

    return k(f0, f1, f2, f3a, f3b, labels2, zc128, ones_px)


def _sc_agg(xblks, src_t, dstc, with_counts):
    nb = len(xblks)
    outs = [jax.ShapeDtypeStruct((NNP, 128), jnp.float32)
            for _ in range(2 * nb)]
    scr = [pltpu.VMEM((ECH,), jnp.int32),
           pltpu.VMEM((ECH,), jnp.int32),
           pltpu.VMEM((ECH, 128), jnp.float32),
           pltpu.VMEM((164, 128), jnp.float32),
           pltpu.VMEM_SHARED((NAB, 128), jnp.float32),
           pltpu.SemaphoreType.DMA]
    if with_counts:
        outs.append(jax.ShapeDtypeStruct((2 * NNP, 128), jnp.float32))
    zeros_h = jnp.zeros((164, 128), jnp.float32)
    ones_h = jnp.ones((ECH, 128), jnp.float32)

    @functools.partial(pl.kernel, mesh=_mesh(), out_type=outs,
                       scratch_types=scr)
    def k(src_h, dst_h, z_h, o_h, *rest):
        xs = rest[:nb]
        rest = rest[nb:]
        aggs = rest[:2 * nb]
        rest = rest[2 * nb:]
        if with_counts:
            cnt_h = rest[0]
            rest = rest[1:]
        (src_v, dst_v, rows_v, zeros_v, acc, sem) = rest
        c = lax.axis_index("c")
        s = lax.axis_index("s")
        pltpu.sync_copy(z_h, zeros_v)

        def zero_acc():
            for q in range(2):
                pltpu.sync_copy(zeros_v, acc.at[pl.ds(s * 328 + q * 164, 164)])

        for h in range(2):
            q16 = (c * 2 + h) * 16
            if with_counts:
                zero_acc()
                pltpu.sync_copy(o_h, rows_v)
                plsc.subcore_barrier()
                for kk in range(NECH):
                    pltpu.sync_copy(dst_h.at[(q16 + s) * NECH + kk], dst_v)
                    pltpu.sync_copy(rows_v, acc.at[dst_v], add=True)
                plsc.subcore_barrier()
                pltpu.sync_copy(
                    acc.at[pl.ds(s * 320, 320)],
                    cnt_h.at[pl.ds(c * NNP + h * NH + s * 320, 320)])
            for fb in range(nb):
                zero_acc()
                plsc.subcore_barrier()
                for kk in range(NECH):
                    pltpu.sync_copy(src_h.at[s * NECH + kk], src_v)
                    pltpu.sync_copy(dst_h.at[(q16 + s) * NECH + kk], dst_v)
                    pltpu.async_copy(xs[fb].at[src_v], rows_v, sem).wait()
                    pltpu.sync_copy(rows_v, acc.at[dst_v], add=True)
                plsc.subcore_barrier()
                for cc in range(2):
                    @pl.when(c == cc)
                    def _():
                        pltpu.sync_copy(
                            acc.at[pl.ds(s * 320, 320)],
                            aggs[2 * fb + cc].at[pl.ds(h * NH + s * 320, 320)])

    return k(src_t, dstc, zeros_h, ones_h, *xblks)


def _norm_pool3(sums0, cnt, sums1, sums2, sums3):
    s0 = sums0.reshape(B, SPAD, 128)
    cn = cnt.reshape(B, SPAD, 128)
    s1 = sums1.reshape(B, SPAD, 128)
    s2 = sums2.reshape(B, SPAD, 128)
    s3 = sums3.reshape(B, SPAD, 256)

    def body(s0_ref, c_ref, s1_ref, s2_ref, s3_ref,
             x0_ref, k1_ref, k2_ref, k3_ref):
        cinv = 1.0 / jnp.maximum(c_ref[0, :, 0], 1.0)
        cinv = cinv[:, None]
        x0 = s0_ref[0] * cinv
        x0_ref[0] = jnp.concatenate(
            [x0[:, :36], jnp.zeros((SPAD, 92), jnp.float32)], axis=1)
        k1_ref[0] = s1_ref[0, :, :64] * cinv
        k2_ref[0] = s2_ref[0] * cinv
        k3_ref[0] = s3_ref[0] * cinv

    def spec(C):
        return pl.BlockSpec((1, SPAD, C), lambda b: (b, 0, 0))

    outs = pl.pallas_call(
        body,
        grid=(B,),
        in_specs=[spec(128), spec(128), spec(128), spec(128), spec(256)],
        out_specs=[spec(128), spec(64), spec(128), spec(256)],
        out_shape=[jax.ShapeDtypeStruct((B, SPAD, 128), jnp.float32),
                   jax.ShapeDtypeStruct((B, SPAD, 64), jnp.float32),
                   jax.ShapeDtypeStruct((B, SPAD, 128), jnp.float32),
                   jax.ShapeDtypeStruct((B, SPAD, 256), jnp.float32)],
    )(s0, cn, s1, s2, s3)
    return [o[:, :SSP, :].reshape(NN, o.shape[2]) for o in outs]


def _lin_layer(xblks, aggPs, aggNs, cP, cN, Wpl, Wpr, bp, Wnl, Wnr, bn_, F,
               layer0=False):
    nb = len(xblks)
    Fo = Wpl.shape[0]

    def body(*refs):
        (xrefs, rest) = (refs[:nb], refs[nb:])
        aPr, rest = rest[:nb], rest[nb:]
        aNr, rest = rest[:nb], rest[nb:]
        (cP_ref, cN_ref, wpl_ref, wpr_ref, bp_ref, wnl_ref, wnr_ref, bn_ref,
         pre_ref, st_ref, acc_ref) = rest
        r = pl.program_id(0)
        x = jnp.concatenate([xr[...] for xr in xrefs], axis=1)
        aP = jnp.concatenate([ar[...] for ar in aPr], axis=1)
        aN = jnp.concatenate([ar[...] for ar in aNr], axis=1)
        cPi = 1.0 / jnp.maximum(cP_ref[...], 1.0)
        cNi = 1.0 / jnp.maximum(cN_ref[...], 1.0)
        aP = aP * cPi
        aN = aN * cNi
        if layer0:
            op = aP[:, :36]
            on = aN[:, :36]
            xp = x[:, :36]
            xn = x[:, :36]
        else:
            op = jnp.concatenate([aP[:, :F], aN[:, F:]], axis=1)
            on = jnp.concatenate([aP[:, F:], aN[:, :F]], axis=1)
            xp = x[:, :F]
            xn = x[:, F:]
        outp = (lax.dot_general(op, wpl_ref[...], (((1,), (1,)), ((), ())),
                                preferred_element_type=jnp.float32)
                + lax.dot_general(xp, wpr_ref[...], (((1,), (1,)), ((), ())),
                                  preferred_element_type=jnp.float32)
                + bp_ref[...])
        outn = (lax.dot_general(on, wnl_ref[...], (((1,), (1,)), ((), ())),
                                preferred_element_type=jnp.float32)
                + lax.dot_general(xn, wnr_ref[...], (((1,), (1,)), ((), ())),
                                  preferred_element_type=jnp.float32)
                + bn_ref[...])
        pre = jnp.concatenate([outp, outn], axis=1)
        pre_ref[...] = pre

        @pl.when(r == 0)
        def _():
            acc_ref[...] = jnp.zeros_like(acc_ref)

        acc_ref[0:1, :] += jnp.sum(pre, axis=0, keepdims=True)
        acc_ref[1:2, :] += jnp.sum(pre * pre, axis=0, keepdims=True)

        @pl.when(r == NRB - 1)
        def _():
            st_ref[...] = acc_ref[...]

    rspec = pl.BlockSpec((RB, 128), lambda r: (r, 0))
    wspec = lambda W: pl.BlockSpec(W.shape, lambda r: (0, 0))
    return pl.pallas_call(
        body,
        grid=(NRB,),
        in_specs=([rspec] * nb + [rspec] * nb + [rspec] * nb
                  + [pl.BlockSpec((RB, 1), lambda r: (r, 0))] * 2
                  + [wspec(Wpl), wspec(Wpr),
                     pl.BlockSpec((1, Fo), lambda r: (0, 0)),
                     wspec(Wnl), wspec(Wnr),
                     pl.BlockSpec((1, Fo), lambda r: (0, 0))]),
        out_specs=[pl.BlockSpec((RB, 2 * Fo), lambda r: (r, 0)),
                   pl.BlockSpec((8, 2 * Fo), lambda r: (0, 0))],
        out_shape=[jax.ShapeDtypeStruct((NN, 2 * Fo), jnp.float32),
                   jax.ShapeDtypeStruct((8, 2 * Fo), jnp.float32)],
        scratch_shapes=[pltpu.VMEM((8, 2 * Fo), jnp.float32)],
    )(*xblks, *aggPs, *aggNs, cP, cN,
      Wpl, Wpr, bp.reshape(1, Fo), Wnl, Wnr, bn_.reshape(1, Fo))


def _bn_assemble(pre, stats, g, b_, skipn, nb_next):
    Fo2 = pre.shape[1]
    Fo = Fo2 // 2
    Csk = skipn.shape[1]

    def body(p_ref, st_ref, g_ref, b_ref, sk_ref, *orefs):
        m = st_ref[0:1, :] / NN
        v = st_ref[1:2, :] / NN - m * m
        scale = g_ref[...] * lax.rsqrt(v + 1e-5)
        y = jax.nn.relu((p_ref[...] - m) * scale + b_ref[...])
        sk = sk_ref[...]
        full = jnp.concatenate([y[:, Fo:], sk, y[:, :Fo], sk], axis=1)
        for i, o in enumerate(orefs):
            o[...] = full[:, i * 128:(i + 1) * 128]

    outs = pl.pallas_call(
        body,
        grid=(NRB,),
        in_specs=[pl.BlockSpec((RB, Fo2), lambda r: (r, 0)),
                  pl.BlockSpec((8, Fo2), lambda r: (0, 0)),
                  pl.BlockSpec((1, Fo2), lambda r: (0, 0)),
                  pl.BlockSpec((1, Fo2), lambda r: (0, 0)),
                  pl.BlockSpec((RB, Csk), lambda r: (r, 0))],
        out_specs=[pl.BlockSpec((RB, 128), lambda r: (r, 0))] * nb_next,
        out_shape=[jax.ShapeDtypeStruct((NN, 128), jnp.float32)] * nb_next,
    )(pre, stats, g.reshape(1, Fo2), b_.reshape(1, Fo2), skipn)
    return outs


def _bn_final(pre, stats, g, b_, Wpw, bpw):
    Fo2 = pre.shape[1]
    Fo = Fo2 // 2

    def body(p_ref, st_ref, g_ref, b_ref, w_ref, bw_ref, o_ref):
        m = st_ref[0:1, :] / NN
        v = st_ref[1:2, :] / NN - m * m
        scale = g_ref[...] * lax.rsqrt(v + 1e-5)
        y = jax.nn.relu((p_ref[...] - m) * scale + b_ref[...])
        xf = jnp.concatenate([y[:, Fo:], y[:, :Fo]], axis=1)
        o_ref[...] = jax.nn.relu(
            lax.dot_general(xf, w_ref[...], (((1,), (1,)), ((), ())),
                            preferred_element_type=jnp.float32) + bw_ref[...])

    return pl.pallas_call(
        body,
        grid=(NRB,),
        in_specs=[pl.BlockSpec((RB, Fo2), lambda r: (r, 0)),
                  pl.BlockSpec((8, Fo2), lambda r: (0, 0)),
                  pl.BlockSpec((1, Fo2), lambda r: (0, 0)),
                  pl.BlockSpec((1, Fo2), lambda r: (0, 0)),
                  pl.BlockSpec(Wpw.shape, lambda r: (0, 0)),
                  pl.BlockSpec((1, Wpw.shape[0]), lambda r: (0, 0))],
        out_specs=pl.BlockSpec((RB, Wpw.shape[0]), lambda r: (r, 0)),
        out_shape=jax.ShapeDtypeStruct((NN, Wpw.shape[0]), jnp.float32),
    )(pre, stats, g.reshape(1, Fo2), b_.reshape(1, Fo2), Wpw,
      bpw.reshape(1, Wpw.shape[0]))


def kernel(labels, edges_nn, fx, fy, skip0, skip1, skip2, skip3,
           W_pl0, W_pr0, b_p0, W_nl0, W_nr0, b_n0,
           W_pl1, W_pr1, b_p1, W_nl1, W_nr1, b_n1,
           W_pl2, W_pr2, b_p2, W_nl2, W_nr2, b_n2,
           W_pl3, W_pr3, b_p3, W_nl3, W_nr3, b_n3,
           bn0_g, bn0_b, bn1_g, bn1_b, bn2_g, bn2_b, bn3_g, bn3_b,
           W_pw, b_pw):
    labels = labels.reshape(B, 16384).astype(jnp.int32)
    edges_nn = edges_nn.astype(jnp.int32)

    s0cm = _upsample(skip0, _up_mat(64))
    s1cm = _upsample(skip1, _up_mat(32))
    s2cm = _upsample(skip2, _up_mat(16))
    s3cm = _upsample(skip3, _up_mat(8))
    f0 = _feat0_pixel_major(s0cm, fx.reshape(B, 1, 16384),
                            fy.reshape(B, 1, 16384))
    f1 = _to_pixel_major(s1cm, 128)
    f2 = _to_pixel_major(s2cm, 128)
    f3a = _to_pixel_major(s3cm[:, :128], 128)
    f3b = _to_pixel_major(s3cm[:, 128:], 128)

    lab2 = _label_prep(labels).reshape(B * 64, PCH)
    sums0, cnt, sums1, sums2, s3s_a, s3s_b = _sc_pool_all(f0, f1, f2, f3a, f3b, lab2)
    sums3 = jnp.concatenate([s3s_a.reshape(B, SPAD, 128), s3s_b.reshape(B, SPAD, 128)], axis=2).reshape(B * SPAD, 256)

    x0blk, skip1n, skip2n, skip3n = _norm_pool3(sums0, cnt, sums1, sums2,
                                                sums3)

    ep = _edge_prep(edges_nn)
    src_t = ep[0].reshape(16 * NECH, ECH)
    dstc = ep[1:5].reshape(4 * 16 * NECH, ECH)

    def split(aggs, nb):
        aP = [aggs[2 * i][:NN] for i in range(nb)]
        aN = [aggs[2 * i + 1][:NN] for i in range(nb)]
        return aP, aN

    *aggs0, ecnt = _sc_agg([x0blk], src_t, dstc, True)
    aP, aN = split(aggs0, 1)
    ec = ecnt.reshape(2, NNP, 128)
    cP = ec[0, :NN, 0:1]
    cN = ec[1, :NN, 0:1]
    pre0, st0 = _lin_layer([x0blk], aP, aN, cP, cN,
                           W_pl0, W_pr0, b_p0, W_nl0, W_nr0, b_n0,
                           36, layer0=True)
    x1b = _bn_assemble(pre0, st0, bn0_g, bn0_b, skip1n, 2)

    aggs = _sc_agg(x1b, src_t, dstc, False)
    aP, aN = split(aggs, 2)
    pre1, st1 = _lin_layer(x1b, aP, aN, cP, cN,
                           W_pl1, W_pr1, b_p1, W_nl1, W_nr1, b_n1, 128)
    x2b = _bn_assemble(pre1, st1, bn1_g, bn1_b, skip2n, 4)

    aggs = _sc_agg(x2b, src_t, dstc, False)
    aP, aN = split(aggs, 4)
    pre2, st2 = _lin_layer(x2b, aP, aN, cP, cN,
                           W_pl2, W_pr2, b_p2, W_nl2, W_nr2, b_n2, 256)
    x3b = _bn_assemble(pre2, st2, bn2_g, bn2_b, skip3n, 8)

    aggs = _sc_agg(x3b, src_t, dstc, False)
    aP, aN = split(aggs, 8)
    pre3, st3 = _lin_layer(x3b, aP, aN, cP, cN,
                           W_pl3, W_pr3, b_p3, W_nl3, W_nr3, b_n3, 512)

    return _bn_final(pre3, st3, bn3_g, bn3_b, W_pw, b_pw)


# speedup vs baseline: 1.8172x; 1.0074x over previous
import functools
import numpy as np
import jax
import jax.numpy as jnp
from jax import lax
from jax.experimental import pallas as pl
from jax.experimental.pallas import tpu as pltpu
from jax.experimental.pallas import tpu_sc as plsc

B = 4
SSP = 2500
SPAD = 2560
NN = 10000
NNP = 10240
NH = 5120
NAB = 5248
TRASH = 5184
EE = 160000
EEP = 163840
EPT = EEP // 16
ECH = 512
NECH = EPT // ECH
PPT = 16384 // 16
PCH = 256
NPCH = PPT // PCH
RB = 1000
NRB = NN // RB


@functools.cache
def _mesh():
    return plsc.VectorSubcoreMesh(core_axis_name="c", subcore_axis_name="s")


def _up_mat(w):
    xs = np.linspace(0.0, w - 1.0, 128)
    x0 = np.floor(xs).astype(np.int32)
    x1 = np.minimum(x0 + 1, w - 1)
    wx = (xs - x0).astype(np.float32)
    U = np.zeros((128, w), np.float32)
    np.add.at(U, (np.arange(128), x0), 1.0 - wx)
    np.add.at(U, (np.arange(128), x1), wx)
    return jnp.asarray(U)


def _upsample(f, U):
    Bb, C, w, h = f.shape

    def body(f_ref, u_ref, o_ref):
        f3 = f_ref[0]
        Um = u_ref[...]
        t1 = lax.dot_general(Um, f3, (((1,), (1,)), ((), ())),
                             preferred_element_type=jnp.float32)
        t2 = lax.dot_general(t1, Um, (((2,), (1,)), ((), ())),
                             preferred_element_type=jnp.float32)
        o_ref[0] = jnp.transpose(t2, (1, 0, 2)).reshape(C, 128 * 128)

    return pl.pallas_call(
        body,
        grid=(Bb,),
        in_specs=[pl.BlockSpec((1, C, w, h), lambda b: (b, 0, 0, 0)),
                  pl.BlockSpec((128, w), lambda b: (0, 0))],
        out_specs=pl.BlockSpec((1, C, 16384), lambda b: (b, 0, 0)),
        out_shape=jax.ShapeDtypeStruct((Bb, C, 16384), jnp.float32),
    )(f, U)


def _to_pixel_major(cm, Cp):
    Bb, C, _ = cm.shape
    PB = 2048

    def body(x_ref, o_ref):
        x = x_ref[0]
        y = jnp.transpose(x, (1, 0))
        if Cp > C:
            y = jnp.concatenate(
                [y, jnp.zeros((PB, Cp - C), jnp.float32)], axis=1)
        o_ref[...] = y

    return pl.pallas_call(
        body,
        grid=(Bb, 16384 // PB),
        in_specs=[pl.BlockSpec((1, C, PB), lambda b, j: (b, 0, j))],
        out_specs=pl.BlockSpec((PB, Cp), lambda b, j: (b * (16384 // PB) + j, 0)),
        out_shape=jax.ShapeDtypeStruct((Bb * 16384, Cp), jnp.float32),
    )(cm)


def _feat0_pixel_major(s0cm, fx, fy):
    PB = 2048

    def body(s_ref, fx_ref, fy_ref, o_ref):
        j = pl.program_id(1)
        s = jnp.transpose(s_ref[0], (1, 0))
        pix = j * PB + lax.broadcasted_iota(jnp.int32, (PB, 1), 0)
        cx = (pix // 128).astype(jnp.float32) / 127.0
        cy = (pix % 128).astype(jnp.float32) / 127.0
        fxc = jnp.transpose(fx_ref[0], (1, 0))
        fyc = jnp.transpose(fy_ref[0], (1, 0))
        y = jnp.concatenate(
            [cx, cy, fxc, fyc, s, jnp.zeros((PB, 92), jnp.float32)], axis=1)
        o_ref[...] = y

    return pl.pallas_call(
        body,
        grid=(B, 16384 // PB),
        in_specs=[pl.BlockSpec((1, 32, PB), lambda b, j: (b, 0, j)),
                  pl.BlockSpec((1, 1, PB), lambda b, j: (b, 0, j)),
                  pl.BlockSpec((1, 1, PB), lambda b, j: (b, 0, j))],
        out_specs=pl.BlockSpec((PB, 128), lambda b, j: (b * (16384 // PB) + j, 0)),
        out_shape=jax.ShapeDtypeStruct((B * 16384, 128), jnp.float32),
    )(s0cm, fx, fy)


def _edge_prep(edges_nn):
    def body(e_ref, o_ref):
        pad = jnp.full((EEP - EE,), TRASH, jnp.int32)
        zpad = jnp.zeros((EEP - EE,), jnp.int32)
        dst = e_ref[1]
        sgn = e_ref[2]
        neg = sgn == -1
        o_ref[0, :] = jnp.concatenate([e_ref[0], zpad])
        for q, (want_neg, h) in enumerate([(False, 0), (False, 1),
                                           (True, 0), (True, 1)]):
            sel = ((neg == want_neg) & (dst >= h * NH) & (dst < (h + 1) * NH))
            loc = jnp.where(sel, dst - h * NH, TRASH)
            o_ref[1 + q, :] = jnp.concatenate([loc, pad])
        o_ref[5, :] = jnp.zeros((EEP,), jnp.int32)
        o_ref[6, :] = jnp.zeros((EEP,), jnp.int32)
        o_ref[7, :] = jnp.zeros((EEP,), jnp.int32)

    return pl.pallas_call(
        body,
        in_specs=[pl.BlockSpec((3, EE), lambda: (0, 0))],
        out_specs=pl.BlockSpec((8, EEP), lambda: (0, 0)),
        out_shape=jax.ShapeDtypeStruct((8, EEP), jnp.int32),
    )(edges_nn)


def _label_prep(labels):
    def body(l_ref, o_ref):
        b = pl.program_id(0)
        o_ref[...] = l_ref[...] + (b % 2) * SPAD

    return pl.pallas_call(
        body,
        grid=(B,),
        in_specs=[pl.BlockSpec((1, 1, 16384), lambda b: (b, 0, 0))],
        out_specs=pl.BlockSpec((1, 1, 16384), lambda b: (b, 0, 0)),
        out_shape=jax.ShapeDtypeStruct((B, 1, 16384), jnp.int32),
    )(labels.reshape(B, 1, 16384))


def _sc_pool_all(f0, f1, f2, f3a, f3b, labels2):
    outs = [jax.ShapeDtypeStruct((B * SPAD, 128), jnp.float32)] * 6
    scr = [pltpu.VMEM((PCH, 128), jnp.float32),
           pltpu.VMEM((PCH,), jnp.int32),
           pltpu.VMEM_SHARED((SPAD, 128), jnp.float32)]
    zc128 = jnp.zeros((160, 128), jnp.float32)
    ones_px = jnp.ones((PCH, 128), jnp.float32)

    @functools.partial(pl.kernel, mesh=_mesh(), out_type=outs,
                       scratch_types=scr)
    def k(f0_h, f1_h, f2_h, f3a_h, f3b_h, lab_h, zc_h, ones_h,
          s0_h, cnt_h, s1_h, s2_h, s3a_h, s3b_h, rows_v, lab_v, acc):
        c = lax.axis_index("c")
        s = lax.axis_index("s")

        def do_set(feats_h, out_h):
            for b in range(2):
                batch = c * 2 + b
                pltpu.sync_copy(zc_h, acc.at[pl.ds(s * 160, 160)])
                plsc.subcore_barrier()
                for kk in range(NPCH):
                    pltpu.sync_copy(lab_h.at[(batch * 16 + s) * NPCH + kk],
                                    lab_v)
                    if feats_h is not None:
                        row0 = batch * 16384 + s * PPT + kk * PCH
                        pltpu.sync_copy(feats_h.at[pl.ds(row0, PCH)], rows_v)
                    pltpu.sync_copy(rows_v, acc.at[lab_v], add=True)
                plsc.subcore_barrier()
                pltpu.sync_copy(acc.at[pl.ds(s * 160, 160)],
                                out_h.at[pl.ds(batch * SPAD + s * 160, 160)])
                plsc.subcore_barrier()

        do_set(f0_h, s0_h)
        pltpu.sync_copy(ones_h, rows_v)
        do_set(None, cnt_h)
        do_set(f1_h, s1_h)
        do_set(f2_h, s2_h)
        do_set(f3a_h, s3a_h)
        do_set(f3b_h, s3b_h)

    return k(f0, f1, f2, f3a, f3b, labels2, zc128, ones_px)


def _sc_agg(xblks, src_t, dstc, with_counts):
    nb = len(xblks)
    outs = [jax.ShapeDtypeStruct((NNP, 128), jnp.float32)
            for _ in range(2 * nb)]
    scr = [pltpu.VMEM((ECH,), jnp.int32),
           pltpu.VMEM((ECH,), jnp.int32),
           pltpu.VMEM((ECH, 128), jnp.float32),
           pltpu.VMEM((164, 128), jnp.float32),
           pltpu.VMEM_SHARED((NAB, 128), jnp.float32),
           pltpu.SemaphoreType.DMA]
    if with_counts:
        outs.append(jax.ShapeDtypeStruct((2 * NNP, 128), jnp.float32))
    zeros_h = jnp.zeros((164, 128), jnp.float32)
    ones_h = jnp.ones((ECH, 128), jnp.float32)

    @functools.partial(pl.kernel, mesh=_mesh(), out_type=outs,
                       scratch_types=scr)
    def k(src_h, dst_h, z_h, o_h, *rest):
        xs = rest[:nb]
        rest = rest[nb:]
        aggs = rest[:2 * nb]
        rest = rest[2 * nb:]
        if with_counts:
            cnt_h = rest[0]
            rest = rest[1:]
        (src_v, dst_v, rows_v, zeros_v, acc, sem) = rest
        c = lax.axis_index("c")
        s = lax.axis_index("s")
        pltpu.sync_copy(z_h, zeros_v)

        def zero_acc():
            for q in range(2):
                pltpu.sync_copy(zeros_v, acc.at[pl.ds(s * 328 + q * 164, 164)])

        for h in range(2):
            q16 = (c * 2 + h) * 16
            if with_counts:
                zero_acc()
                pltpu.sync_copy(o_h, rows_v)
                plsc.subcore_barrier()
                for kk in range(NECH):
                    pltpu.sync_copy(dst_h.at[(q16 + s) * NECH + kk], dst_v)
                    pltpu.sync_copy(rows_v, acc.at[dst_v], add=True)
                plsc.subcore_barrier()
                pltpu.sync_copy(
                    acc.at[pl.ds(s * 320, 320)],
                    cnt_h.at[pl.ds(c * NNP + h * NH + s * 320, 320)])
            for fb in range(nb):
                zero_acc()
                plsc.subcore_barrier()
                for kk in range(NECH):
                    pltpu.sync_copy(src_h.at[s * NECH + kk], src_v)
                    pltpu.sync_copy(dst_h.at[(q16 + s) * NECH + kk], dst_v)
                    pltpu.async_copy(xs[fb].at[src_v], rows_v, sem).wait()
                    pltpu.sync_copy(rows_v, acc.at[dst_v], add=True)
                plsc.subcore_barrier()
                for cc in range(2):
                    @pl.when(c == cc)
                    def _():
                        pltpu.sync_copy(
                            acc.at[pl.ds(s * 320, 320)],
                            aggs[2 * fb + cc].at[pl.ds(h * NH + s * 320, 320)])

    return k(src_t, dstc, zeros_h, ones_h, *xblks)


def _norm_pool3(sums0, cnt, sums1, sums2, sums3):
    s0 = sums0.reshape(B, SPAD, 128)
    cn = cnt.reshape(B, SPAD, 128)
    s1 = sums1.reshape(B, SPAD, 128)
    s2 = sums2.reshape(B, SPAD, 128)
    s3 = sums3.reshape(B, SPAD, 256)

    def body(s0_ref, c_ref, s1_ref, s2_ref, s3_ref,
             x0_ref, k1_ref, k2_ref, k3_ref):
        cinv = 1.0 / jnp.maximum(c_ref[0, :, 0], 1.0)
        cinv = cinv[:, None]
        x0 = s0_ref[0] * cinv
        x0_ref[0] = jnp.concatenate(
            [x0[:, :36], jnp.zeros((SPAD, 92), jnp.float32)], axis=1)
        k1_ref[0] = s1_ref[0, :, :64] * cinv
        k2_ref[0] = s2_ref[0] * cinv
        k3_ref[0] = s3_ref[0] * cinv

    def spec(C):
        return pl.BlockSpec((1, SPAD, C), lambda b: (b, 0, 0))

    outs = pl.pallas_call(
        body,
        grid=(B,),
        in_specs=[spec(128), spec(128), spec(128), spec(128), spec(256)],
        out_specs=[spec(128), spec(64), spec(128), spec(256)],
        out_shape=[jax.ShapeDtypeStruct((B, SPAD, 128), jnp.float32),
                   jax.ShapeDtypeStruct((B, SPAD, 64), jnp.float32),
                   jax.ShapeDtypeStruct((B, SPAD, 128), jnp.float32),
                   jax.ShapeDtypeStruct((B, SPAD, 256), jnp.float32)],
    )(s0, cn, s1, s2, s3)
    return [o[:, :SSP, :].reshape(NN, o.shape[2]) for o in outs]


def _lin_layer(xblks, aggPs, aggNs, cP, cN, Wpl, Wpr, bp, Wnl, Wnr, bn_, F,
               layer0=False):
    nb = len(xblks)
    Fo = Wpl.shape[0]

    def body(*refs):
        (xrefs, rest) = (refs[:nb], refs[nb:])
        aPr, rest = rest[:nb], rest[nb:]
        aNr, rest = rest[:nb], rest[nb:]
        (cP_ref, cN_ref, wpl_ref, wpr_ref, bp_ref, wnl_ref, wnr_ref, bn_ref,
         pre_ref, st_ref, acc_ref) = rest
        r = pl.program_id(0)
        x = jnp.concatenate([xr[...] for xr in xrefs], axis=1)
        aP = jnp.concatenate([ar[...] for ar in aPr], axis=1)
        aN = jnp.concatenate([ar[...] for ar in aNr], axis=1)
        cPi = 1.0 / jnp.maximum(cP_ref[...], 1.0)
        cNi = 1.0 / jnp.maximum(cN_ref[...], 1.0)
        aP = aP * cPi
        aN = aN * cNi
        if layer0:
            op = aP[:, :36]
            on = aN[:, :36]
            xp = x[:, :36]
            xn = x[:, :36]
        else:
            op = jnp.concatenate([aP[:, :F], aN[:, F:]], axis=1)
            on = jnp.concatenate([aP[:, F:], aN[:, :F]], axis=1)
            xp = x[:, :F]
            xn = x[:, F:]
        outp = (lax.dot_general(op, wpl_ref[...], (((1,), (1,)), ((), ())),
                                preferred_element_type=jnp.float32)
                + lax.dot_general(xp, wpr_ref[...], (((1,), (1,)), ((), ())),
                                  preferred_element_type=jnp.float32)
                + bp_ref[...])
        outn = (lax.dot_general(on, wnl_ref[...], (((1,), (1,)), ((), ())),
                                preferred_element_type=jnp.float32)
                + lax.dot_general(xn, wnr_ref[...], (((1,), (1,)), ((), ())),
                                  preferred_element_type=jnp.float32)
                + bn_ref[...])
        pre = jnp.concatenate([outp, outn], axis=1)
        pre_ref[...] = pre

        @pl.when(r == 0)
        def _():
            acc_ref[...] = jnp.zeros_like(acc_ref)

        acc_ref[0:1, :] += jnp.sum(pre, axis=0, keepdims=True)
        acc_ref[1:2, :] += jnp.sum(pre * pre, axis=0, keepdims=True)

        @pl.when(r == NRB - 1)
        def _():
            st_ref[...] = acc_ref[...]

    rspec = pl.BlockSpec((RB, 128), lambda r: (r, 0))
    wspec = lambda W: pl.BlockSpec(W.shape, lambda r: (0, 0))
    return pl.pallas_call(
        body,
        grid=(NRB,),
        in_specs=([rspec] * nb + [rspec] * nb + [rspec] * nb
                  + [pl.BlockSpec((RB, 1), lambda r: (r, 0))] * 2
                  + [wspec(Wpl), wspec(Wpr),
                     pl.BlockSpec((1, Fo), lambda r: (0, 0)),
                     wspec(Wnl), wspec(Wnr),
                     pl.BlockSpec((1, Fo), lambda r: (0, 0))]),
        out_specs=[pl.BlockSpec((RB, 2 * Fo), lambda r: (r, 0)),
                   pl.BlockSpec((8, 2 * Fo), lambda r: (0, 0))],
        out_shape=[jax.ShapeDtypeStruct((NN, 2 * Fo), jnp.float32),
                   jax.ShapeDtypeStruct((8, 2 * Fo), jnp.float32)],
        scratch_shapes=[pltpu.VMEM((8, 2 * Fo), jnp.float32)],
    )(*xblks, *aggPs, *aggNs, cP, cN,
      Wpl, Wpr, bp.reshape(1, Fo), Wnl, Wnr, bn_.reshape(1, Fo))


def _bn_assemble(pre, stats, g, b_, skipn, nb_next):
    Fo2 = pre.shape[1]
    Fo = Fo2 // 2
    Csk = skipn.shape[1]

    def body(p_ref, st_ref, g_ref, b_ref, sk_ref, *orefs):
        m = st_ref[0:1, :] / NN
        v = st_ref[1:2, :] / NN - m * m
        scale = g_ref[...] * lax.rsqrt(v + 1e-5)
        y = jax.nn.relu((p_ref[...] - m) * scale + b_ref[...])
        sk = sk_ref[...]
        full = jnp.concatenate([y[:, Fo:], sk, y[:, :Fo], sk], axis=1)
        for i, o in enumerate(orefs):
            o[...] = full[:, i * 128:(i + 1) * 128]

    outs = pl.pallas_call(
        body,
        grid=(NRB,),
        in_specs=[pl.BlockSpec((RB, Fo2), lambda r: (r, 0)),
                  pl.BlockSpec((8, Fo2), lambda r: (0, 0)),
                  pl.BlockSpec((1, Fo2), lambda r: (0, 0)),
                  pl.BlockSpec((1, Fo2), lambda r: (0, 0)),
                  pl.BlockSpec((RB, Csk), lambda r: (r, 0))],
        out_specs=[pl.BlockSpec((RB, 128), lambda r: (r, 0))] * nb_next,
        out_shape=[jax.ShapeDtypeStruct((NN, 128), jnp.float32)] * nb_next,
    )(pre, stats, g.reshape(1, Fo2), b_.reshape(1, Fo2), skipn)
    return outs


def _bn_final(pre, stats, g, b_, Wpw, bpw):
    Fo2 = pre.shape[1]
    Fo = Fo2 // 2

    def body(p_ref, st_ref, g_ref, b_ref, w_ref, bw_ref, o_ref):
        m = st_ref[0:1, :] / NN
        v = st_ref[1:2, :] / NN - m * m
        scale = g_ref[...] * lax.rsqrt(v + 1e-5)
        y = jax.nn.relu((p_ref[...] - m) * scale + b_ref[...])
        xf = jnp.concatenate([y[:, Fo:], y[:, :Fo]], axis=1)
        o_ref[...] = jax.nn.relu(
            lax.dot_general(xf, w_ref[...], (((1,), (1,)), ((), ())),
                            preferred_element_type=jnp.float32) + bw_ref[...])

    return pl.pallas_call(
        body,
        grid=(NRB,),
        in_specs=[pl.BlockSpec((RB, Fo2), lambda r: (r, 0)),
                  pl.BlockSpec((8, Fo2), lambda r: (0, 0)),
                  pl.BlockSpec((1, Fo2), lambda r: (0, 0)),
                  pl.BlockSpec((1, Fo2), lambda r: (0, 0)),
                  pl.BlockSpec(Wpw.shape, lambda r: (0, 0)),
                  pl.BlockSpec((1, Wpw.shape[0]), lambda r: (0, 0))],
        out_specs=pl.BlockSpec((RB, Wpw.shape[0]), lambda r: (r, 0)),
        out_shape=jax.ShapeDtypeStruct((NN, Wpw.shape[0]), jnp.float32),
    )(pre, stats, g.reshape(1, Fo2), b_.reshape(1, Fo2), Wpw,
      bpw.reshape(1, Wpw.shape[0]))


def kernel(labels, edges_nn, fx, fy, skip0, skip1, skip2, skip3,
           W_pl0, W_pr0, b_p0, W_nl0, W_nr0, b_n0,
           W_pl1, W_pr1, b_p1, W_nl1, W_nr1, b_n1,
           W_pl2, W_pr2, b_p2, W_nl2, W_nr2, b_n2,
           W_pl3, W_pr3, b_p3, W_nl3, W_nr3, b_n3,
           bn0_g, bn0_b, bn1_g, bn1_b, bn2_g, bn2_b, bn3_g, bn3_b,
           W_pw, b_pw):
    labels = labels.reshape(B, 16384).astype(jnp.int32)
    edges_nn = edges_nn.astype(jnp.int32)

    s0cm = _upsample(skip0, _up_mat(64))
    s1cm = _upsample(skip1, _up_mat(32))
    s2cm = _upsample(skip2, _up_mat(16))
    s3cm = _upsample(skip3, _up_mat(8))
    f0 = _feat0_pixel_major(s0cm, fx.reshape(B, 1, 16384),
                            fy.reshape(B, 1, 16384))
    f1 = _to_pixel_major(s1cm, 128)
    f2 = _to_pixel_major(s2cm, 128)
    f3a = _to_pixel_major(s3cm[:, :128], 128)
    f3b = _to_pixel_major(s3cm[:, 128:], 128)

    lab2 = labels.reshape(B * 64, PCH)
    sums0, cnt, sums1, sums2, s3s_a, s3s_b = _sc_pool_all(f0, f1, f2, f3a, f3b, lab2)
    sums3 = jnp.concatenate([s3s_a.reshape(B, SPAD, 128), s3s_b.reshape(B, SPAD, 128)], axis=2).reshape(B * SPAD, 256)

    x0blk, skip1n, skip2n, skip3n = _norm_pool3(sums0, cnt, sums1, sums2,
                                                sums3)

    ep = _edge_prep(edges_nn)
    src_t = ep[0].reshape(16 * NECH, ECH)
    dstc = ep[1:5].reshape(4 * 16 * NECH, ECH)

    def split(aggs, nb):
        aP = [aggs[2 * i][:NN] for i in range(nb)]
        aN = [aggs[2 * i + 1][:NN] for i in range(nb)]
        return aP, aN

    *aggs0, ecnt = _sc_agg([x0blk], src_t, dstc, True)
    aP, aN = split(aggs0, 1)
    ec = ecnt.reshape(2, NNP, 128)
    cP = ec[0, :NN, 0:1]
    cN = ec[1, :NN, 0:1]
    pre0, st0 = _lin_layer([x0blk], aP, aN, cP, cN,
                           W_pl0, W_pr0, b_p0, W_nl0, W_nr0, b_n0,
                           36, layer0=True)
    x1b = _bn_assemble(pre0, st0, bn0_g, bn0_b, skip1n, 2)

    aggs = _sc_agg(x1b, src_t, dstc, False)
    aP, aN = split(aggs, 2)
    pre1, st1 = _lin_layer(x1b, aP, aN, cP, cN,
                           W_pl1, W_pr1, b_p1, W_nl1, W_nr1, b_n1, 128)
    x2b = _bn_assemble(pre1, st1, bn1_g, bn1_b, skip2n, 4)

    aggs = _sc_agg(x2b, src_t, dstc, False)
    aP, aN = split(aggs, 4)
    pre2, st2 = _lin_layer(x2b, aP, aN, cP, cN,
                           W_pl2, W_pr2, b_p2, W_nl2, W_nr2, b_n2, 256)
    x3b = _bn_assemble(pre2, st2, bn2_g, bn2_b, skip3n, 8)

    aggs = _sc_agg(x3b, src_t, dstc, False)
    aP, aN = split(aggs, 8)
    pre3, st3 = _lin_layer(x3b, aP, aN, cP, cN,
                           W_pl3, W_pr3, b_p3, W_nl3, W_nr3, b_n3, 512)

    return _bn_final(pre3, st3, bn3_g, bn3_b, W_pw, b_pw)


# speedup vs baseline: 1.8444x; 1.0150x over previous
import functools
import numpy as np
import jax
import jax.numpy as jnp
from jax import lax
from jax.experimental import pallas as pl
from jax.experimental.pallas import tpu as pltpu
from jax.experimental.pallas import tpu_sc as plsc

B = 4
SSP = 2500
SPAD = 2560
NN = 10000
NNP = 10240
NH = 5120
NAB = 5248
TRASH = 5184
EE = 160000
EEP = 163840
EPT = EEP // 16
ECH = 320
NECH = EPT // ECH
PPT = 16384 // 16
PCH = 256
NPCH = PPT // PCH
RB = 1000
NRB = NN // RB


@functools.cache
def _mesh():
    return plsc.VectorSubcoreMesh(core_axis_name="c", subcore_axis_name="s")


def _up_mat(w):
    xs = np.linspace(0.0, w - 1.0, 128)
    x0 = np.floor(xs).astype(np.int32)
    x1 = np.minimum(x0 + 1, w - 1)
    wx = (xs - x0).astype(np.float32)
    U = np.zeros((128, w), np.float32)
    np.add.at(U, (np.arange(128), x0), 1.0 - wx)
    np.add.at(U, (np.arange(128), x1), wx)
    return jnp.asarray(U)


def _upsample(f, U):
    Bb, C, w, h = f.shape

    def body(f_ref, u_ref, o_ref):
        f3 = f_ref[0]
        Um = u_ref[...]
        t1 = lax.dot_general(Um, f3, (((1,), (1,)), ((), ())),
                             preferred_element_type=jnp.float32)
        t2 = lax.dot_general(t1, Um, (((2,), (1,)), ((), ())),
                             preferred_element_type=jnp.float32)
        o_ref[0] = jnp.transpose(t2, (1, 0, 2)).reshape(C, 128 * 128)

    return pl.pallas_call(
        body,
        grid=(Bb,),
        in_specs=[pl.BlockSpec((1, C, w, h), lambda b: (b, 0, 0, 0)),
                  pl.BlockSpec((128, w), lambda b: (0, 0))],
        out_specs=pl.BlockSpec((1, C, 16384), lambda b: (b, 0, 0)),
        out_shape=jax.ShapeDtypeStruct((Bb, C, 16384), jnp.float32),
    )(f, U)


def _to_pixel_major(cm, Cp):
    Bb, C, _ = cm.shape
    PB = 2048

    def body(x_ref, o_ref):
        x = x_ref[0]
        y = jnp.transpose(x, (1, 0))
        if Cp > C:
            y = jnp.concatenate(
                [y, jnp.zeros((PB, Cp - C), jnp.float32)], axis=1)
        o_ref[...] = y

    return pl.pallas_call(
        body,
        grid=(Bb, 16384 // PB),
        in_specs=[pl.BlockSpec((1, C, PB), lambda b, j: (b, 0, j))],
        out_specs=pl.BlockSpec((PB, Cp), lambda b, j: (b * (16384 // PB) + j, 0)),
        out_shape=jax.ShapeDtypeStruct((Bb * 16384, Cp), jnp.float32),
    )(cm)


def _feat0_pixel_major(s0cm, fx, fy):
    PB = 2048

    def body(s_ref, fx_ref, fy_ref, o_ref):
        j = pl.program_id(1)
        s = jnp.transpose(s_ref[0], (1, 0))
        pix = j * PB + lax.broadcasted_iota(jnp.int32, (PB, 1), 0)
        cx = (pix // 128).astype(jnp.float32) / 127.0
        cy = (pix % 128).astype(jnp.float32) / 127.0
        fxc = jnp.transpose(fx_ref[0], (1, 0))
        fyc = jnp.transpose(fy_ref[0], (1, 0))
        y = jnp.concatenate(
            [cx, cy, fxc, fyc, s, jnp.zeros((PB, 92), jnp.float32)], axis=1)
        o_ref[...] = y

    return pl.pallas_call(
        body,
        grid=(B, 16384 // PB),
        in_specs=[pl.BlockSpec((1, 32, PB), lambda b, j: (b, 0, j)),
                  pl.BlockSpec((1, 1, PB), lambda b, j: (b, 0, j)),
                  pl.BlockSpec((1, 1, PB), lambda b, j: (b, 0, j))],
        out_specs=pl.BlockSpec((PB, 128), lambda b, j: (b * (16384 // PB) + j, 0)),
        out_shape=jax.ShapeDtypeStruct((B * 16384, 128), jnp.float32),
    )(s0cm, fx, fy)


def _edge_prep(edges_nn):
    def body(e_ref, o_ref):
        pad = jnp.full((EEP - EE,), TRASH, jnp.int32)
        zpad = jnp.zeros((EEP - EE,), jnp.int32)
        dst = e_ref[1]
        sgn = e_ref[2]
        neg = sgn == -1
        o_ref[0, :] = jnp.concatenate([e_ref[0], zpad])
        for q, (want_neg, h) in enumerate([(False, 0), (False, 1),
                                           (True, 0), (True, 1)]):
            sel = ((neg == want_neg) & (dst >= h * NH) & (dst < (h + 1) * NH))
            loc = jnp.where(sel, dst - h * NH, TRASH)
            o_ref[1 + q, :] = jnp.concatenate([loc, pad])
        o_ref[5, :] = jnp.zeros((EEP,), jnp.int32)
        o_ref[6, :] = jnp.zeros((EEP,), jnp.int32)
        o_ref[7, :] = jnp.zeros((EEP,), jnp.int32)

    return pl.pallas_call(
        body,
        in_specs=[pl.BlockSpec((3, EE), lambda: (0, 0))],
        out_specs=pl.BlockSpec((8, EEP), lambda: (0, 0)),
        out_shape=jax.ShapeDtypeStruct((8, EEP), jnp.int32),
    )(edges_nn)


def _label_prep(labels):
    def body(l_ref, o_ref):
        b = pl.program_id(0)
        o_ref[...] = l_ref[...] + (b % 2) * SPAD

    return pl.pallas_call(
        body,
        grid=(B,),
        in_specs=[pl.BlockSpec((1, 1, 16384), lambda b: (b, 0, 0))],
        out_specs=pl.BlockSpec((1, 1, 16384), lambda b: (b, 0, 0)),
        out_shape=jax.ShapeDtypeStruct((B, 1, 16384), jnp.int32),
    )(labels.reshape(B, 1, 16384))


def _sc_pool_all(f0, f1, f2, f3a, f3b, labels2):
    outs = [jax.ShapeDtypeStruct((B * SPAD, 128), jnp.float32)] * 6
    scr = [pltpu.VMEM((PCH, 128), jnp.float32),
           pltpu.VMEM((PCH,), jnp.int32),
           pltpu.VMEM_SHARED((SPAD, 128), jnp.float32)]
    zc128 = jnp.zeros((160, 128), jnp.float32)
    ones_px = jnp.ones((PCH, 128), jnp.float32)

    @functools.partial(pl.kernel, mesh=_mesh(), out_type=outs,
                       scratch_types=scr)
    def k(f0_h, f1_h, f2_h, f3a_h, f3b_h, lab_h, zc_h, ones_h,
          s0_h, cnt_h, s1_h, s2_h, s3a_h, s3b_h, rows_v, lab_v, acc):
        c = lax.axis_index("c")
        s = lax.axis_index("s")

        def do_set(feats_h, out_h):
            for b in range(2):
                batch = c * 2 + b
                pltpu.sync_copy(zc_h, acc.at[pl.ds(s * 160, 160)])
                plsc.subcore_barrier()
                for kk in range(NPCH):
                    pltpu.sync_copy(lab_h.at[(batch * 16 + s) * NPCH + kk],
                                    lab_v)
                    if feats_h is not None:
                        row0 = batch * 16384 + s * PPT + kk * PCH
                        pltpu.sync_copy(feats_h.at[pl.ds(row0, PCH)], rows_v)
                    pltpu.sync_copy(rows_v, acc.at[lab_v], add=True)
                plsc.subcore_barrier()
                pltpu.sync_copy(acc.at[pl.ds(s * 160, 160)],
                                out_h.at[pl.ds(batch * SPAD + s * 160, 160)])
                plsc.subcore_barrier()

        do_set(f0_h, s0_h)
        pltpu.sync_copy(ones_h, rows_v)
        do_set(None, cnt_h)
        do_set(f1_h, s1_h)
        do_set(f2_h, s2_h)
        do_set(f3a_h, s3a_h)
        do_set(f3b_h, s3b_h)

    return k(f0, f1, f2, f3a, f3b, labels2, zc128, ones_px)


def _sc_agg(xblks, src_t, dstc, with_counts):
    nb = len(xblks)
    outs = [jax.ShapeDtypeStruct((NNP, 128), jnp.float32)
            for _ in range(2 * nb)]
    scr = [pltpu.VMEM((ECH,), jnp.int32),
           pltpu.VMEM((ECH,), jnp.int32),
           pltpu.VMEM((ECH,), jnp.int32),
           pltpu.VMEM((ECH,), jnp.int32),
           pltpu.VMEM((ECH, 128), jnp.float32),
           pltpu.VMEM((ECH, 128), jnp.float32),
           pltpu.VMEM_SHARED((NAB, 128), jnp.float32),
           pltpu.SemaphoreType.DMA]
    if with_counts:
        outs.append(jax.ShapeDtypeStruct((2 * NNP, 128), jnp.float32))
    zeros_h = jnp.zeros((328, 128), jnp.float32)
    ones_h = jnp.ones((ECH, 128), jnp.float32)

    @functools.partial(pl.kernel, mesh=_mesh(), out_type=outs,
                       scratch_types=scr)
    def k(src_h, dst_h, z_h, o_h, *rest):
        xs = rest[:nb]
        rest = rest[nb:]
        aggs = rest[:2 * nb]
        rest = rest[2 * nb:]
        if with_counts:
            cnt_h = rest[0]
            rest = rest[1:]
        (src_v0, src_v1, dst_v0, dst_v1, rows_v0, rows_v1,
         acc, sem) = rest
        c = lax.axis_index("c")
        s = lax.axis_index("s")
        def zero_acc():
            pltpu.sync_copy(z_h, acc.at[pl.ds(s * 328, 328)])

        for h in range(2):
            q16 = (c * 2 + h) * 16
            if with_counts:
                zero_acc()
                pltpu.sync_copy(o_h, rows_v0)
                plsc.subcore_barrier()
                for kk in range(NECH):
                    pltpu.sync_copy(dst_h.at[(q16 + s) * NECH + kk], dst_v0)
                    pltpu.sync_copy(rows_v0, acc.at[dst_v0], add=True)
                plsc.subcore_barrier()
                pltpu.sync_copy(
                    acc.at[pl.ds(s * 320, 320)],
                    cnt_h.at[pl.ds(c * NNP + h * NH + s * 320, 320)])
            for fb in range(nb):
                zero_acc()
                plsc.subcore_barrier()
                pltpu.sync_copy(src_h.at[s * NECH], src_v0)
                pltpu.sync_copy(dst_h.at[(q16 + s) * NECH], dst_v0)

                def chunk_pair(i, carry):
                    kk = 2 * i
                    g0 = pltpu.async_copy(xs[fb].at[src_v0], rows_v0, sem)
                    pltpu.sync_copy(src_h.at[s * NECH + kk + 1], src_v1)
                    pltpu.sync_copy(dst_h.at[(q16 + s) * NECH + kk + 1],
                                    dst_v1)
                    g1 = pltpu.async_copy(xs[fb].at[src_v1], rows_v1, sem)
                    g0.wait()
                    pltpu.sync_copy(rows_v0, acc.at[dst_v0], add=True)
                    nxt = jnp.minimum(kk + 2, NECH - 2)
                    pltpu.sync_copy(src_h.at[s * NECH + nxt], src_v0)
                    pltpu.sync_copy(dst_h.at[(q16 + s) * NECH + nxt], dst_v0)
                    g1.wait()
                    pltpu.sync_copy(rows_v1, acc.at[dst_v1], add=True)
                    return carry

                lax.fori_loop(0, NECH // 2, chunk_pair, 0, unroll=False)
                plsc.subcore_barrier()
                for cc in range(2):
                    @pl.when(c == cc)
                    def _():
                        pltpu.sync_copy(
                            acc.at[pl.ds(s * 320, 320)],
                            aggs[2 * fb + cc].at[pl.ds(h * NH + s * 320, 320)])

    return k(src_t, dstc, zeros_h, ones_h, *xblks)


def _norm_pool3(sums0, cnt, sums1, sums2, sums3):
    s0 = sums0.reshape(B, SPAD, 128)
    cn = cnt.reshape(B, SPAD, 128)
    s1 = sums1.reshape(B, SPAD, 128)
    s2 = sums2.reshape(B, SPAD, 128)
    s3 = sums3.reshape(B, SPAD, 256)

    def body(s0_ref, c_ref, s1_ref, s2_ref, s3_ref,
             x0_ref, k1_ref, k2_ref, k3_ref):
        cinv = 1.0 / jnp.maximum(c_ref[0, :, 0], 1.0)
        cinv = cinv[:, None]
        x0 = s0_ref[0] * cinv
        x0_ref[0] = jnp.concatenate(
            [x0[:, :36], jnp.zeros((SPAD, 92), jnp.float32)], axis=1)
        k1_ref[0] = s1_ref[0, :, :64] * cinv
        k2_ref[0] = s2_ref[0] * cinv
        k3_ref[0] = s3_ref[0] * cinv

    def spec(C):
        return pl.BlockSpec((1, SPAD, C), lambda b: (b, 0, 0))

    outs = pl.pallas_call(
        body,
        grid=(B,),
        in_specs=[spec(128), spec(128), spec(128), spec(128), spec(256)],
        out_specs=[spec(128), spec(64), spec(128), spec(256)],
        out_shape=[jax.ShapeDtypeStruct((B, SPAD, 128), jnp.float32),
                   jax.ShapeDtypeStruct((B, SPAD, 64), jnp.float32),
                   jax.ShapeDtypeStruct((B, SPAD, 128), jnp.float32),
                   jax.ShapeDtypeStruct((B, SPAD, 256), jnp.float32)],
    )(s0, cn, s1, s2, s3)
    return [o[:, :SSP, :].reshape(NN, o.shape[2]) for o in outs]


def _lin_layer(xblks, aggPs, aggNs, cP, cN, Wpl, Wpr, bp, Wnl, Wnr, bn_, F,
               layer0=False):
    nb = len(xblks)
    Fo = Wpl.shape[0]

    def body(*refs):
        (xrefs, rest) = (refs[:nb], refs[nb:])
        aPr, rest = rest[:nb], rest[nb:]
        aNr, rest = rest[:nb], rest[nb:]
        (cP_ref, cN_ref, wpl_ref, wpr_ref, bp_ref, wnl_ref, wnr_ref, bn_ref,
         pre_ref, st_ref, acc_ref) = rest
        r = pl.program_id(0)
        x = jnp.concatenate([xr[...] for xr in xrefs], axis=1)
        aP = jnp.concatenate([ar[...] for ar in aPr], axis=1)
        aN = jnp.concatenate([ar[...] for ar in aNr], axis=1)
        cPi = 1.0 / jnp.maximum(cP_ref[...], 1.0)
        cNi = 1.0 / jnp.maximum(cN_ref[...], 1.0)
        aP = aP * cPi
        aN = aN * cNi
        if layer0:
            op = aP[:, :36]
            on = aN[:, :36]
            xp = x[:, :36]
            xn = x[:, :36]
        else:
            op = jnp.concatenate([aP[:, :F], aN[:, F:]], axis=1)
            on = jnp.concatenate([aP[:, F:], aN[:, :F]], axis=1)
            xp = x[:, :F]
            xn = x[:, F:]
        outp = (lax.dot_general(op, wpl_ref[...], (((1,), (1,)), ((), ())),
                                preferred_element_type=jnp.float32)
                + lax.dot_general(xp, wpr_ref[...], (((1,), (1,)), ((), ())),
                                  preferred_element_type=jnp.float32)
                + bp_ref[...])
        outn = (lax.dot_general(on, wnl_ref[...], (((1,), (1,)), ((), ())),
                                preferred_element_type=jnp.float32)
                + lax.dot_general(xn, wnr_ref[...], (((1,), (1,)), ((), ())),
                                  preferred_element_type=jnp.float32)
                + bn_ref[...])
        pre = jnp.concatenate([outp, outn], axis=1)
        pre_ref[...] = pre

        @pl.when(r == 0)
        def _():
            acc_ref[...] = jnp.zeros_like(acc_ref)

        acc_ref[0:1, :] += jnp.sum(pre, axis=0, keepdims=True)
        acc_ref[1:2, :] += jnp.sum(pre * pre, axis=0, keepdims=True)

        @pl.when(r == NRB - 1)
        def _():
            st_ref[...] = acc_ref[...]

    rspec = pl.BlockSpec((RB, 128), lambda r: (r, 0))
    wspec = lambda W: pl.BlockSpec(W.shape, lambda r: (0, 0))
    return pl.pallas_call(
        body,
        grid=(NRB,),
        in_specs=([rspec] * nb + [rspec] * nb + [rspec] * nb
                  + [pl.BlockSpec((RB, 1), lambda r: (r, 0))] * 2
                  + [wspec(Wpl), wspec(Wpr),
                     pl.BlockSpec((1, Fo), lambda r: (0, 0)),
                     wspec(Wnl), wspec(Wnr),
                     pl.BlockSpec((1, Fo), lambda r: (0, 0))]),
        out_specs=[pl.BlockSpec((RB, 2 * Fo), lambda r: (r, 0)),
                   pl.BlockSpec((8, 2 * Fo), lambda r: (0, 0))],
        out_shape=[jax.ShapeDtypeStruct((NN, 2 * Fo), jnp.float32),
                   jax.ShapeDtypeStruct((8, 2 * Fo), jnp.float32)],
        scratch_shapes=[pltpu.VMEM((8, 2 * Fo), jnp.float32)],
    )(*xblks, *aggPs, *aggNs, cP, cN,
      Wpl, Wpr, bp.reshape(1, Fo), Wnl, Wnr, bn_.reshape(1, Fo))


def _bn_assemble(pre, stats, g, b_, skipn, nb_next):
    Fo2 = pre.shape[1]
    Fo = Fo2 // 2
    Csk = skipn.shape[1]

    def body(p_ref, st_ref, g_ref, b_ref, sk_ref, *orefs):
        m = st_ref[0:1, :] / NN
        v = st_ref[1:2, :] / NN - m * m
        scale = g_ref[...] * lax.rsqrt(v + 1e-5)
        y = jax.nn.relu((p_ref[...] - m) * scale + b_ref[...])
        sk = sk_ref[...]
        full = jnp.concatenate([y[:, Fo:], sk, y[:, :Fo], sk], axis=1)
        for i, o in enumerate(orefs):
            o[...] = full[:, i * 128:(i + 1) * 128]

    outs = pl.pallas_call(
        body,
        grid=(NRB,),
        in_specs=[pl.BlockSpec((RB, Fo2), lambda r: (r, 0)),
                  pl.BlockSpec((8, Fo2), lambda r: (0, 0)),
                  pl.BlockSpec((1, Fo2), lambda r: (0, 0)),
                  pl.BlockSpec((1, Fo2), lambda r: (0, 0)),
                  pl.BlockSpec((RB, Csk), lambda r: (r, 0))],
        out_specs=[pl.BlockSpec((RB, 128), lambda r: (r, 0))] * nb_next,
        out_shape=[jax.ShapeDtypeStruct((NN, 128), jnp.float32)] * nb_next,
    )(pre, stats, g.reshape(1, Fo2), b_.reshape(1, Fo2), skipn)
    return outs


def _bn_final(pre, stats, g, b_, Wpw, bpw):
    Fo2 = pre.shape[1]
    Fo = Fo2 // 2

    def body(p_ref, st_ref, g_ref, b_ref, w_ref, bw_ref, o_ref):
        m = st_ref[0:1, :] / NN
        v = st_ref[1:2, :] / NN - m * m
        scale = g_ref[...] * lax.rsqrt(v + 1e-5)
        y = jax.nn.relu((p_ref[...] - m) * scale + b_ref[...])
        xf = jnp.concatenate([y[:, Fo:], y[:, :Fo]], axis=1)
        o_ref[...] = jax.nn.relu(
            lax.dot_general(xf, w_ref[...], (((1,), (1,)), ((), ())),
                            preferred_element_type=jnp.float32) + bw_ref[...])

    return pl.pallas_call(
        body,
        grid=(NRB,),
        in_specs=[pl.BlockSpec((RB, Fo2), lambda r: (r, 0)),
                  pl.BlockSpec((8, Fo2), lambda r: (0, 0)),
                  pl.BlockSpec((1, Fo2), lambda r: (0, 0)),
                  pl.BlockSpec((1, Fo2), lambda r: (0, 0)),
                  pl.BlockSpec(Wpw.shape, lambda r: (0, 0)),
                  pl.BlockSpec((1, Wpw.shape[0]), lambda r: (0, 0))],
        out_specs=pl.BlockSpec((RB, Wpw.shape[0]), lambda r: (r, 0)),
        out_shape=jax.ShapeDtypeStruct((NN, Wpw.shape[0]), jnp.float32),
    )(pre, stats, g.reshape(1, Fo2), b_.reshape(1, Fo2), Wpw,
      bpw.reshape(1, Wpw.shape[0]))


def kernel(labels, edges_nn, fx, fy, skip0, skip1, skip2, skip3,
           W_pl0, W_pr0, b_p0, W_nl0, W_nr0, b_n0,
           W_pl1, W_pr1, b_p1, W_nl1, W_nr1, b_n1,
           W_pl2, W_pr2, b_p2, W_nl2, W_nr2, b_n2,
           W_pl3, W_pr3, b_p3, W_nl3, W_nr3, b_n3,
           bn0_g, bn0_b, bn1_g, bn1_b, bn2_g, bn2_b, bn3_g, bn3_b,
           W_pw, b_pw):
    labels = labels.reshape(B, 16384).astype(jnp.int32)
    edges_nn = edges_nn.astype(jnp.int32)

    s0cm = _upsample(skip0, _up_mat(64))
    s1cm = _upsample(skip1, _up_mat(32))
    s2cm = _upsample(skip2, _up_mat(16))
    s3cm = _upsample(skip3, _up_mat(8))
    f0 = _feat0_pixel_major(s0cm, fx.reshape(B, 1, 16384),
                            fy.reshape(B, 1, 16384))
    f1 = _to_pixel_major(s1cm, 128)
    f2 = _to_pixel_major(s2cm, 128)
    f3a = _to_pixel_major(s3cm[:, :128], 128)
    f3b = _to_pixel_major(s3cm[:, 128:], 128)

    lab2 = labels.reshape(B * 64, PCH)
    sums0, cnt, sums1, sums2, s3s_a, s3s_b = _sc_pool_all(f0, f1, f2, f3a, f3b, lab2)
    sums3 = jnp.concatenate([s3s_a.reshape(B, SPAD, 128), s3s_b.reshape(B, SPAD, 128)], axis=2).reshape(B * SPAD, 256)

    x0blk, skip1n, skip2n, skip3n = _norm_pool3(sums0, cnt, sums1, sums2,
                                                sums3)

    ep = _edge_prep(edges_nn)
    src_t = ep[0].reshape(16 * NECH, ECH)
    dstc = ep[1:5].reshape(4 * 16 * NECH, ECH)

    def split(aggs, nb):
        aP = [aggs[2 * i][:NN] for i in range(nb)]
        aN = [aggs[2 * i + 1][:NN] for i in range(nb)]
        return aP, aN

    *aggs0, ecnt = _sc_agg([x0blk], src_t, dstc, True)
    aP, aN = split(aggs0, 1)
    ec = ecnt.reshape(2, NNP, 128)
    cP = ec[0, :NN, 0:1]
    cN = ec[1, :NN, 0:1]
    pre0, st0 = _lin_layer([x0blk], aP, aN, cP, cN,
                           W_pl0, W_pr0, b_p0, W_nl0, W_nr0, b_n0,
                           36, layer0=True)
    x1b = _bn_assemble(pre0, st0, bn0_g, bn0_b, skip1n, 2)

    aggs = _sc_agg(x1b, src_t, dstc, False)
    aP, aN = split(aggs, 2)
    pre1, st1 = _lin_layer(x1b, aP, aN, cP, cN,
                           W_pl1, W_pr1, b_p1, W_nl1, W_nr1, b_n1, 128)
    x2b = _bn_assemble(pre1, st1, bn1_g, bn1_b, skip2n, 4)

    aggs = _sc_agg(x2b, src_t, dstc, False)
    aP, aN = split(aggs, 4)
    pre2, st2 = _lin_layer(x2b, aP, aN, cP, cN,
                           W_pl2, W_pr2, b_p2, W_nl2, W_nr2, b_n2, 256)
    x3b = _bn_assemble(pre2, st2, bn2_g, bn2_b, skip3n, 8)

    aggs = _sc_agg(x3b, src_t, dstc, False)
    aP, aN = split(aggs, 8)
    pre3, st3 = _lin_layer(x3b, aP, aN, cP, cN,
                           W_pl3, W_pr3, b_p3, W_nl3, W_nr3, b_n3, 512)

    return _bn_final(pre3, st3, bn3_g, bn3_b, W_pw, b_pw)
